# baseline (speedup 1.0000x reference)
import functools
import jax
import jax.numpy as jnp
from jax import lax
from jax.experimental import pallas as pl
from jax.experimental.pallas import tpu as pltpu
from jax.experimental.pallas import tpu_sc as plsc

NUM_ENT = 50000
NUM_REL = 500
NUM_STEPS = 3
DIM = 768
BSZ = 2
E = 16000
K = 4
L_Q = 31

N_TILES = 16
ENT_PAD = 50176
ENT_SL = ENT_PAD // N_TILES
REL_PAD = 512
E_PAD = 16384
EPT = E_PAD // N_TILES
NJ = EPT // 128
ROW_BLK = 3584


def _tc_prep_body(te_ref, qw_ref, sw_ref, sb_ref,
                  rcw_ref, ecw_ref, tcw_ref, scal_ref,
                  hw_ref, hb_ref, aw_ref, ab_ref,
                  vte_ref, vtr_ref, se_ref, sr_ref, be_ref, br_ref, wv_ref):
    f32 = jnp.float32
    te = te_ref[...]
    rcw = rcw_ref[...]
    ecw = ecw_ref[...]
    tcw = tcw_ref[...]
    scal = scal_ref[...]
    rel_cls_b, ent_cls_b, type_cls_b = scal[0, 0], scal[0, 1], scal[0, 2]

    vte_ref[...] = jnp.zeros((8, DIM), f32)
    vtr_ref[...] = jnp.zeros((8, DIM), f32)
    se_ref[...] = jnp.zeros((8, 1), f32)
    sr_ref[...] = jnp.zeros((8, 1), f32)
    be_ref[...] = jnp.full((8, 1), ent_cls_b, f32)
    br_ref[...] = jnp.full((8, 1), rel_cls_b, f32)

    for t in range(NUM_STEPS):
        w = sw_ref[t]
        cq = jnp.tanh(
            jnp.dot(te, w, preferred_element_type=f32) + sb_ref[pl.ds(t, 1)])
        for b in range(BSZ):
            cqb = cq[b:b + 1, :]
            qwb = qw_ref[pl.ds(b * L_Q, L_Q), :]
            logit = jnp.dot(qwb, cqb.T, preferred_element_type=f32)
            m = jnp.max(logit, axis=0, keepdims=True)
            e = jnp.exp(logit - m)
            d = e / jnp.sum(e, axis=0, keepdims=True)
            d = d / (jnp.sum(d, axis=0, keepdims=True) + 1e-6)
            ctx = jnp.dot(d.T, qwb, preferred_element_type=f32) + cqb
            r = b * NUM_STEPS + t
            ts = jax.nn.sigmoid(jnp.sum(ctx * tcw, axis=1, keepdims=True)
                                + type_cls_b)
            vte_ref[pl.ds(r, 1), :] = ctx * ecw
            vtr_ref[pl.ds(r, 1), :] = ctx * rcw
            se_ref[pl.ds(r, 1), :] = 1.0 - ts
            sr_ref[pl.ds(r, 1), :] = ts

    hop = jax.nn.softmax(jnp.dot(te, hw_ref[...],
                                 preferred_element_type=f32) + hb_ref[...],
                         axis=1)
    ans = jax.nn.softmax(jnp.dot(te, aw_ref[...],
                                 preferred_element_type=f32) + ab_ref[...],
                         axis=1)
    went = hop * ans[:, 0:1]
    wrel = hop * ans[:, 1:2]
    wv_ref[...] = jnp.concatenate(
        [went, wrel, jnp.zeros((BSZ, 16 - 2 * NUM_STEPS), f32)], axis=1)


def _tc_prob_body(vt_ref, scale_ref, bias_ref, x_ref, o_ref):
    x = x_ref[...]
    vt = vt_ref[...]
    logits = jnp.einsum("kd,nd->kn", vt, x,
                        preferred_element_type=jnp.float32)
    logits = logits + bias_ref[...]
    o_ref[...] = scale_ref[...] * jax.nn.sigmoid(logits)


def _tc_prob_tables(table, vt, scale, bias, n_out):
    n_rows = table.shape[0]
    blk = min(ROW_BLK, n_out)
    grid = n_out // blk
    assert grid * blk == n_out and n_out >= n_rows
    return pl.pallas_call(
        _tc_prob_body,
        grid=(grid,),
        in_specs=[
            pl.BlockSpec((8, DIM), lambda i: (0, 0)),
            pl.BlockSpec((8, 1), lambda i: (0, 0)),
            pl.BlockSpec((8, 1), lambda i: (0, 0)),
            pl.BlockSpec((blk, DIM), lambda i: (i, 0)),
        ],
        out_specs=pl.BlockSpec((8, blk), lambda i: (0, i)),
        out_shape=jax.ShapeDtypeStruct((8, n_out), jnp.float32),
    )(vt, scale, bias, table)


def _sc_body(eterm_h, rterm_h, sub_h, rel_h, obj_h, valid_h,
             keyi_h, keyv_h, wv_h,
             ent_out, rel_out,
             zero_v, sub_iv, rel_iv, obj_iv, valid_v,
             ss_v, ep_v, rp_v, contrib_v, tprob_v,
             rwork_v, frel_v,
             eslice_v, norm_v, fent_v,
             wv_v, keyi_v, keyv_v, padd_v,
             gsem, ssem,
             laste_sp, enew_sp, eterm0_sp, eterm1_sp, eterm2_sp,
             rterm0_sp, rterm1_sp, rterm2_sp,
             rnew0_sp, rnew1_sp, partials0_sp, partials1_sp):
    eterm_sps = (eterm0_sp, eterm1_sp, eterm2_sp)
    rterm_sps = (rterm0_sp, rterm1_sp, rterm2_sp)
    rnew_sps = (rnew0_sp, rnew1_sp)
    partials_sps = (partials0_sp, partials1_sp)
    cid = lax.axis_index("c")
    tid = lax.axis_index("s")
    b = cid
    ent_lo = tid * ENT_SL
    lane = lax.iota(jnp.int32, 16)

    def al8(x):
        return pl.multiple_of(x, 8)

    def hsum(v):
        s = v[0]
        for i in range(1, 16):
            s = s + v[i]
        return s

    edge_descs = [
        pltpu.async_copy(sub_h.at[b, pl.ds(tid * NJ, NJ), :], sub_iv, gsem),
        pltpu.async_copy(rel_h.at[b, pl.ds(tid * NJ, NJ), :], rel_iv, gsem),
        pltpu.async_copy(obj_h.at[b, pl.ds(tid * NJ, NJ), :], obj_iv, gsem),
        pltpu.async_copy(valid_h.at[b, pl.ds(tid * NJ, NJ), :], valid_v,
                         gsem),
        pltpu.async_copy(wv_h.at[pl.ds(al8(b * 16), 16)], wv_v, gsem),
    ]
    bounce = (eslice_v, norm_v, fent_v)
    et_descs = [
        pltpu.async_copy(
            eterm_h.at[pl.ds(al8((b * NUM_STEPS + t) * ENT_PAD + ent_lo),
                             ENT_SL)],
            bounce[t], ssem)
        for t in range(NUM_STEPS)
    ]

    def _zz(i, _):
        zero_v[pl.ds(i * 16, 16)] = jnp.zeros((16,), jnp.float32)
        return 0
    lax.fori_loop(0, ENT_SL // 16, _zz, 0)

    for t in range(NUM_STEPS):
        et_descs[t].wait()
        pltpu.sync_copy(bounce[t], eterm_sps[t].at[pl.ds(ent_lo, ENT_SL)])

    def _z196(i, _):
        fent_v[pl.ds(i * 16, 16)] = jnp.zeros((16,), jnp.float32)
        return 0
    lax.fori_loop(0, ENT_SL // 16, _z196, 0)

    def _z32(i, _):
        frel_v[pl.ds(i * 16, 16)] = jnp.zeros((16,), jnp.float32)
        return 0
    lax.fori_loop(0, REL_PAD // 16, _z32, 0)

    pltpu.sync_copy(zero_v, laste_sp.at[pl.ds(ent_lo, ENT_SL)])
    pltpu.sync_copy(zero_v, enew_sp.at[pl.ds(ent_lo, ENT_SL)])

    @pl.when(tid == 0)
    def _():
        for p in range(2):
            pltpu.sync_copy(zero_v.at[pl.ds(0, REL_PAD)], rnew_sps[p])
            pltpu.sync_copy(zero_v.at[pl.ds(0, 16)], partials_sps[p])
        for t in range(NUM_STEPS):
            pltpu.sync_copy(
                rterm_h.at[pl.ds(al8((b * NUM_STEPS + t) * REL_PAD),
                                 REL_PAD)],
                rwork_v)
            pltpu.sync_copy(rwork_v, rterm_sps[t])

    for d in edge_descs:
        d.wait()
    plsc.subcore_barrier()

    @pl.when(tid == 0)
    def _():
        pltpu.sync_copy(keyi_h.at[pl.ds(al8(b * 16), 16)], keyi_v)
        pltpu.sync_copy(keyv_h.at[pl.ds(al8(b * 16), 16)], keyv_v)
        pltpu.sync_copy(keyv_v, laste_sp.at[keyi_v], add=True)

    margin = jnp.float32(1.0 / K - 1e-6)

    for t in range(NUM_STEPS):
        p = t % 2
        plsc.subcore_barrier()

        g_descs = []
        for j in range(NJ):
            g_descs.append(pltpu.async_copy(
                laste_sp.at[sub_iv.at[j]], ss_v.at[j], gsem))
            g_descs.append(pltpu.async_copy(
                eterm_sps[t].at[obj_iv.at[j]], ep_v.at[j], gsem))
            g_descs.append(pltpu.async_copy(
                rterm_sps[t].at[rel_iv.at[j]], rp_v.at[j], gsem))
        for d in g_descs:
            d.wait()

        def _chunk(c, _):
            j = c // 8
            sl = pl.ds((c % 8) * 16, 16)
            rp = rp_v[j, sl]
            ss16 = ss_v[j, sl]
            ep16 = ep_v[j, sl]
            va = valid_v[j, sl]
            tp = (rp + ep16) * va
            tp = jnp.where(ss16 > margin, tp, jnp.float32(0.0))
            tprob_v[j, sl] = tp
            contrib_v[j, sl] = ss16 * tp
            return 0
        lax.fori_loop(0, EPT // 16, _chunk, 0)

        s_descs = []
        for j in range(NJ):
            s_descs.append(pltpu.async_copy(
                contrib_v.at[j], enew_sp.at[obj_iv.at[j]], ssem, add=True))
            s_descs.append(pltpu.async_copy(
                tprob_v.at[j], rnew_sps[p].at[rel_iv.at[j]], ssem, add=True))
        for d in s_descs:
            d.wait()

        plsc.subcore_barrier()

        rw_d = pltpu.async_copy(rnew_sps[p], rwork_v, gsem)
        pltpu.sync_copy(enew_sp.at[pl.ds(ent_lo, ENT_SL)], eslice_v)

        def _acc(i, a):
            return a + jnp.abs(eslice_v[pl.ds(i * 16, 16)])
        acc = lax.fori_loop(0, ENT_SL // 16, _acc,
                            jnp.zeros((16,), jnp.float32))
        partial = hsum(acc)
        padd_v[...] = jnp.where(lane == tid, partial, jnp.float32(0.0))
        pltpu.sync_copy(padd_v, partials_sps[p].at[lane], add=True)

        pltpu.sync_copy(zero_v, enew_sp.at[pl.ds(ent_lo, ENT_SL)])
        rw_d.wait()
        def _racc(i, a):
            return a + jnp.abs(rwork_v[pl.ds(i * 16, 16)])
        racc = lax.fori_loop(0, REL_PAD // 16, _racc,
                             jnp.zeros((16,), jnp.float32))
        tot_r = jnp.maximum(hsum(racc), jnp.float32(1e-12))

        plsc.subcore_barrier()

        tot_e = hsum(partials_sp_read(wv_v, partials_sps[p], padd_v))
        tot_e = jnp.maximum(tot_e, jnp.float32(1e-12))

        wvec = wv_v[...]
        went = wvec[t]
        wrel = wvec[NUM_STEPS + t]

        def _norm_e(i, _):
            sl = pl.ds(i * 16, 16)
            c = eslice_v[sl] / tot_e
            norm_v[sl] = c
            fent_v[sl] = fent_v[sl] + went * c
            return 0
        lax.fori_loop(0, ENT_SL // 16, _norm_e, 0)

        def _norm_r(i, _):
            sl = pl.ds(i * 16, 16)
            frel_v[sl] = frel_v[sl] + wrel * (rwork_v[sl] / tot_r)
            return 0
        lax.fori_loop(0, REL_PAD // 16, _norm_r, 0)

        pltpu.sync_copy(norm_v, laste_sp.at[pl.ds(ent_lo, ENT_SL)])

        @pl.when(tid == 0)
        def _():
            pltpu.sync_copy(zero_v.at[pl.ds(0, REL_PAD)], rnew_sps[1 - p])
            pltpu.sync_copy(zero_v.at[pl.ds(0, 16)], partials_sps[1 - p])

    pltpu.sync_copy(fent_v, ent_out.at[pl.ds(al8(b * ENT_PAD + ent_lo),
                                             ENT_SL)])

    @pl.when(tid == 0)
    def _():
        pltpu.sync_copy(frel_v, rel_out.at[pl.ds(al8(b * REL_PAD), REL_PAD)])


def partials_sp_read(wv_v, partials_sp, padd_v):
    pltpu.sync_copy(partials_sp, padd_v)
    return padd_v[...]


def _sc_traverse(eterm, rterm, sub_p, rel_p, obj_p, valid_p, keyi, keyv, wv):
    mesh = plsc.VectorSubcoreMesh(core_axis_name="c", subcore_axis_name="s")
    f32, i32 = jnp.float32, jnp.int32
    scratch = [
        pltpu.VMEM((ENT_SL,), f32),
        pltpu.VMEM((NJ, 128), i32),
        pltpu.VMEM((NJ, 128), i32),
        pltpu.VMEM((NJ, 128), i32),
        pltpu.VMEM((NJ, 128), f32),
        pltpu.VMEM((NJ, 128), f32),
        pltpu.VMEM((NJ, 128), f32),
        pltpu.VMEM((NJ, 128), f32),
        pltpu.VMEM((NJ, 128), f32),
        pltpu.VMEM((NJ, 128), f32),
        pltpu.VMEM((REL_PAD,), f32),
        pltpu.VMEM((REL_PAD,), f32),
        pltpu.VMEM((ENT_SL,), f32),
        pltpu.VMEM((ENT_SL,), f32),
        pltpu.VMEM((ENT_SL,), f32),
        pltpu.VMEM((16,), f32),
        pltpu.VMEM((16,), i32),
        pltpu.VMEM((16,), f32),
        pltpu.VMEM((16,), f32),
        pltpu.SemaphoreType.DMA,
        pltpu.SemaphoreType.DMA,
        pltpu.VMEM_SHARED((ENT_PAD,), f32),
        pltpu.VMEM_SHARED((ENT_PAD,), f32),
        pltpu.VMEM_SHARED((ENT_PAD,), f32),
        pltpu.VMEM_SHARED((ENT_PAD,), f32),
        pltpu.VMEM_SHARED((ENT_PAD,), f32),
        pltpu.VMEM_SHARED((REL_PAD,), f32),
        pltpu.VMEM_SHARED((REL_PAD,), f32),
        pltpu.VMEM_SHARED((REL_PAD,), f32),
        pltpu.VMEM_SHARED((REL_PAD,), f32),
        pltpu.VMEM_SHARED((REL_PAD,), f32),
        pltpu.VMEM_SHARED((16,), f32),
        pltpu.VMEM_SHARED((16,), f32),
    ]
    out_type = (
        jax.ShapeDtypeStruct((BSZ * ENT_PAD,), f32),
        jax.ShapeDtypeStruct((BSZ * REL_PAD,), f32),
    )
    ent_out, rel_out = pl.kernel(
        _sc_body, out_type=out_type, mesh=mesh, scratch_types=scratch,
    )(eterm.reshape(-1), rterm.reshape(-1), sub_p, rel_p, obj_p, valid_p,
      keyi.reshape(-1), keyv.reshape(-1), wv.reshape(-1))
    return ent_out.reshape(BSZ, ENT_PAD), rel_out.reshape(BSZ, REL_PAD)


@jax.jit
def kernel(text_emb, q_word, edge_sub, edge_rel, edge_obj, keyconcepts,
           ent_table, rel_table, step_W, step_b,
           rel_cls_w, rel_cls_b, ent_cls_w, ent_cls_b,
           type_cls_w, type_cls_b, hop_w, hop_b, anstype_w, anstype_b):
    f32 = jnp.float32

    full = lambda shape: pl.BlockSpec(shape, lambda: tuple(0 for _ in shape))
    scal = jnp.stack([rel_cls_b, ent_cls_b, type_cls_b,
                      jnp.float32(0.0)]).reshape(1, 4)
    vt_e, vt_r, scale_e, scale_r, bias_e, bias_r, wv = pl.pallas_call(
        _tc_prep_body,
        in_specs=[full((BSZ, DIM)), full((BSZ * L_Q, DIM)),
                  full((NUM_STEPS, DIM, DIM)), full((NUM_STEPS, DIM)),
                  full((1, DIM)), full((1, DIM)), full((1, DIM)),
                  full((1, 4)),
                  full((DIM, NUM_STEPS)), full((1, NUM_STEPS)),
                  full((DIM, 2)), full((1, 2))],
        out_specs=[full((8, DIM)), full((8, DIM)), full((8, 1)),
                   full((8, 1)), full((8, 1)), full((8, 1)),
                   full((BSZ, 16))],
        out_shape=[jax.ShapeDtypeStruct((8, DIM), f32),
                   jax.ShapeDtypeStruct((8, DIM), f32),
                   jax.ShapeDtypeStruct((8, 1), f32),
                   jax.ShapeDtypeStruct((8, 1), f32),
                   jax.ShapeDtypeStruct((8, 1), f32),
                   jax.ShapeDtypeStruct((8, 1), f32),
                   jax.ShapeDtypeStruct((BSZ, 16), f32)],
    )(text_emb, q_word.reshape(BSZ * L_Q, DIM), step_W, step_b,
      rel_cls_w.reshape(1, DIM), ent_cls_w.reshape(1, DIM),
      type_cls_w.reshape(1, DIM), scal,
      hop_w, hop_b.reshape(1, NUM_STEPS), anstype_w, anstype_b.reshape(1, 2))

    eterm = _tc_prob_tables(ent_table, vt_e, scale_e, bias_e, ENT_PAD)
    rterm = _tc_prob_tables(rel_table, vt_r, scale_r, bias_r, REL_PAD)

    i32 = jnp.int32
    def pad_edges(x):
        return jnp.pad(x.astype(i32), ((0, 0), (0, E_PAD - E))
                       ).reshape(BSZ, E_PAD // 128, 128)
    sub_p = pad_edges(edge_sub)
    rel_p = pad_edges(edge_rel)
    obj_p = pad_edges(edge_obj)
    valid = (jnp.arange(E_PAD) < E).astype(f32)
    valid_p = jnp.broadcast_to(valid, (BSZ, E_PAD)).reshape(
        BSZ, E_PAD // 128, 128)

    keyi = jnp.pad(keyconcepts.astype(i32), ((0, 0), (0, 16 - K)))
    keyv = jnp.broadcast_to(
        jnp.where(jnp.arange(16) < K, jnp.float32(1.0 / K), 0.0), (BSZ, 16))

    ent_out, rel_out = _sc_traverse(eterm, rterm, sub_p, rel_p, obj_p,
                                    valid_p, keyi, keyv, wv)

    return jnp.concatenate([ent_out[:, :NUM_ENT], rel_out[:, :NUM_REL]],
                           axis=1)


# speedup vs baseline: 1.0254x; 1.0254x over previous
import functools
import jax
import jax.numpy as jnp
from jax import lax
from jax.experimental import pallas as pl
from jax.experimental.pallas import tpu as pltpu
from jax.experimental.pallas import tpu_sc as plsc

NUM_ENT = 50000
NUM_REL = 500
NUM_STEPS = 3
DIM = 768
BSZ = 2
E = 16000
K = 4

N_TILES = 16
ENT_PAD = 50176
ENT_SL = ENT_PAD // N_TILES
REL_PAD = 512
E_PAD = 16384
EPT = E_PAD // N_TILES
NJ = EPT // 128
ROW_BLK = 3584


def _tc_prob_body(vt_ref, scale_ref, bias_ref, x_ref, o_ref):
    x = x_ref[...]
    vt = vt_ref[...]
    logits = jnp.einsum("kd,nd->kn", vt, x,
                        preferred_element_type=jnp.float32)
    logits = logits + bias_ref[...]
    o_ref[...] = scale_ref[...] * jax.nn.sigmoid(logits)


def _tc_prob_tables(table, vt, scale, bias, n_out):
    n_rows = table.shape[0]
    blk = min(ROW_BLK, n_out)
    grid = n_out // blk
    assert grid * blk == n_out and n_out >= n_rows
    return pl.pallas_call(
        _tc_prob_body,
        grid=(grid,),
        in_specs=[
            pl.BlockSpec((8, DIM), lambda i: (0, 0)),
            pl.BlockSpec((8, 1), lambda i: (0, 0)),
            pl.BlockSpec((8, 1), lambda i: (0, 0)),
            pl.BlockSpec((blk, DIM), lambda i: (i, 0)),
        ],
        out_specs=pl.BlockSpec((8, blk), lambda i: (0, i)),
        out_shape=jax.ShapeDtypeStruct((8, n_out), jnp.float32),
    )(vt, scale, bias, table)


def _sc_body(eterm_h, rterm_h, sub_h, rel_h, obj_h, valid_h,
             keyi_h, keyv_h, wv_h,
             ent_out, rel_out,
             zero_v, sub_iv, rel_iv, obj_iv, valid_v,
             ss_v, ep_v, rp_v, contrib_v, tprob_v,
             rwork_v, frel_v,
             eslice_v, norm_v, fent_v,
             wv_v, keyi_v, keyv_v, padd_v,
             gsem, ssem,
             laste_sp, enew_sp, eterm0_sp, eterm1_sp, eterm2_sp,
             rterm0_sp, rterm1_sp, rterm2_sp,
             rnew0_sp, rnew1_sp, partials0_sp, partials1_sp):
    eterm_sps = (eterm0_sp, eterm1_sp, eterm2_sp)
    rterm_sps = (rterm0_sp, rterm1_sp, rterm2_sp)
    rnew_sps = (rnew0_sp, rnew1_sp)
    partials_sps = (partials0_sp, partials1_sp)
    cid = lax.axis_index("c")
    tid = lax.axis_index("s")
    b = cid
    ent_lo = tid * ENT_SL
    lane = lax.iota(jnp.int32, 16)

    def al8(x):
        return pl.multiple_of(x, 8)

    def hsum(v):
        s = v[0]
        for i in range(1, 16):
            s = s + v[i]
        return s

    edge_descs = [
        pltpu.async_copy(sub_h.at[b, pl.ds(tid * NJ, NJ), :], sub_iv, gsem),
        pltpu.async_copy(rel_h.at[b, pl.ds(tid * NJ, NJ), :], rel_iv, gsem),
        pltpu.async_copy(obj_h.at[b, pl.ds(tid * NJ, NJ), :], obj_iv, gsem),
        pltpu.async_copy(valid_h.at[b, pl.ds(tid * NJ, NJ), :], valid_v,
                         gsem),
        pltpu.async_copy(wv_h.at[pl.ds(al8(b * 16), 16)], wv_v, gsem),
    ]
    bounce = (eslice_v, norm_v, fent_v)
    et_descs = [
        pltpu.async_copy(
            eterm_h.at[pl.ds(al8((b * NUM_STEPS + t) * ENT_PAD + ent_lo),
                             ENT_SL)],
            bounce[t], ssem)
        for t in range(NUM_STEPS)
    ]

    def _zz(i, _):
        zero_v[pl.ds(i * 16, 16)] = jnp.zeros((16,), jnp.float32)
        return 0
    lax.fori_loop(0, ENT_SL // 16, _zz, 0)

    for t in range(NUM_STEPS):
        et_descs[t].wait()
        pltpu.sync_copy(bounce[t], eterm_sps[t].at[pl.ds(ent_lo, ENT_SL)])

    def _z196(i, _):
        fent_v[pl.ds(i * 16, 16)] = jnp.zeros((16,), jnp.float32)
        return 0
    lax.fori_loop(0, ENT_SL // 16, _z196, 0)

    def _z32(i, _):
        frel_v[pl.ds(i * 16, 16)] = jnp.zeros((16,), jnp.float32)
        return 0
    lax.fori_loop(0, REL_PAD // 16, _z32, 0)

    pltpu.sync_copy(zero_v, laste_sp.at[pl.ds(ent_lo, ENT_SL)])
    pltpu.sync_copy(zero_v, enew_sp.at[pl.ds(ent_lo, ENT_SL)])

    @pl.when(tid == 0)
    def _():
        for p in range(2):
            pltpu.sync_copy(zero_v.at[pl.ds(0, REL_PAD)], rnew_sps[p])
            pltpu.sync_copy(zero_v.at[pl.ds(0, 16)], partials_sps[p])
        for t in range(NUM_STEPS):
            pltpu.sync_copy(
                rterm_h.at[pl.ds(al8((b * NUM_STEPS + t) * REL_PAD),
                                 REL_PAD)],
                rwork_v)
            pltpu.sync_copy(rwork_v, rterm_sps[t])

    for d in edge_descs:
        d.wait()
    plsc.subcore_barrier()

    @pl.when(tid == 0)
    def _():
        pltpu.sync_copy(keyi_h.at[pl.ds(al8(b * 16), 16)], keyi_v)
        pltpu.sync_copy(keyv_h.at[pl.ds(al8(b * 16), 16)], keyv_v)
        pltpu.sync_copy(keyv_v, laste_sp.at[keyi_v], add=True)

    margin = jnp.float32(1.0 / K - 1e-6)

    for t in range(NUM_STEPS):
        p = t % 2
        plsc.subcore_barrier()

        g_descs = []
        for j in range(NJ):
            g_descs.append(pltpu.async_copy(
                laste_sp.at[sub_iv.at[j]], ss_v.at[j], gsem))
            g_descs.append(pltpu.async_copy(
                eterm_sps[t].at[obj_iv.at[j]], ep_v.at[j], gsem))
            g_descs.append(pltpu.async_copy(
                rterm_sps[t].at[rel_iv.at[j]], rp_v.at[j], gsem))
        for d in g_descs:
            d.wait()

        def _chunk(c, _):
            j = c // 8
            sl = pl.ds((c % 8) * 16, 16)
            rp = rp_v[j, sl]
            ss16 = ss_v[j, sl]
            ep16 = ep_v[j, sl]
            va = valid_v[j, sl]
            tp = (rp + ep16) * va
            tp = jnp.where(ss16 > margin, tp, jnp.float32(0.0))
            tprob_v[j, sl] = tp
            contrib_v[j, sl] = ss16 * tp
            return 0
        lax.fori_loop(0, EPT // 16, _chunk, 0)

        s_descs = []
        for j in range(NJ):
            s_descs.append(pltpu.async_copy(
                contrib_v.at[j], enew_sp.at[obj_iv.at[j]], ssem, add=True))
            s_descs.append(pltpu.async_copy(
                tprob_v.at[j], rnew_sps[p].at[rel_iv.at[j]], ssem, add=True))
        for d in s_descs:
            d.wait()

        plsc.subcore_barrier()

        rw_d = pltpu.async_copy(rnew_sps[p], rwork_v, gsem)
        pltpu.sync_copy(enew_sp.at[pl.ds(ent_lo, ENT_SL)], eslice_v)

        def _acc(i, a):
            return a + jnp.abs(eslice_v[pl.ds(i * 16, 16)])
        acc = lax.fori_loop(0, ENT_SL // 16, _acc,
                            jnp.zeros((16,), jnp.float32))
        partial = hsum(acc)
        padd_v[...] = jnp.where(lane == tid, partial, jnp.float32(0.0))
        pltpu.sync_copy(padd_v, partials_sps[p].at[lane], add=True)

        pltpu.sync_copy(zero_v, enew_sp.at[pl.ds(ent_lo, ENT_SL)])
        rw_d.wait()
        def _racc(i, a):
            return a + jnp.abs(rwork_v[pl.ds(i * 16, 16)])
        racc = lax.fori_loop(0, REL_PAD // 16, _racc,
                             jnp.zeros((16,), jnp.float32))
        tot_r = jnp.maximum(hsum(racc), jnp.float32(1e-12))

        plsc.subcore_barrier()

        tot_e = hsum(partials_sp_read(wv_v, partials_sps[p], padd_v))
        tot_e = jnp.maximum(tot_e, jnp.float32(1e-12))

        wvec = wv_v[...]
        went = wvec[t]
        wrel = wvec[NUM_STEPS + t]

        def _norm_e(i, _):
            sl = pl.ds(i * 16, 16)
            c = eslice_v[sl] / tot_e
            norm_v[sl] = c
            fent_v[sl] = fent_v[sl] + went * c
            return 0
        lax.fori_loop(0, ENT_SL // 16, _norm_e, 0)

        def _norm_r(i, _):
            sl = pl.ds(i * 16, 16)
            frel_v[sl] = frel_v[sl] + wrel * (rwork_v[sl] / tot_r)
            return 0
        lax.fori_loop(0, REL_PAD // 16, _norm_r, 0)

        pltpu.sync_copy(norm_v, laste_sp.at[pl.ds(ent_lo, ENT_SL)])

        @pl.when(tid == 0)
        def _():
            pltpu.sync_copy(zero_v.at[pl.ds(0, REL_PAD)], rnew_sps[1 - p])
            pltpu.sync_copy(zero_v.at[pl.ds(0, 16)], partials_sps[1 - p])

    pltpu.sync_copy(fent_v, ent_out.at[pl.ds(al8(b * ENT_PAD + ent_lo),
                                             ENT_SL)])

    @pl.when(tid == 0)
    def _():
        pltpu.sync_copy(frel_v, rel_out.at[pl.ds(al8(b * REL_PAD), REL_PAD)])


def partials_sp_read(wv_v, partials_sp, padd_v):
    pltpu.sync_copy(partials_sp, padd_v)
    return padd_v[...]


def _sc_traverse(eterm, rterm, sub_p, rel_p, obj_p, valid_p, keyi, keyv, wv):
    mesh = plsc.VectorSubcoreMesh(core_axis_name="c", subcore_axis_name="s")
    f32, i32 = jnp.float32, jnp.int32
    scratch = [
        pltpu.VMEM((ENT_SL,), f32),
        pltpu.VMEM((NJ, 128), i32),
        pltpu.VMEM((NJ, 128), i32),
        pltpu.VMEM((NJ, 128), i32),
        pltpu.VMEM((NJ, 128), f32),
        pltpu.VMEM((NJ, 128), f32),
        pltpu.VMEM((NJ, 128), f32),
        pltpu.VMEM((NJ, 128), f32),
        pltpu.VMEM((NJ, 128), f32),
        pltpu.VMEM((NJ, 128), f32),
        pltpu.VMEM((REL_PAD,), f32),
        pltpu.VMEM((REL_PAD,), f32),
        pltpu.VMEM((ENT_SL,), f32),
        pltpu.VMEM((ENT_SL,), f32),
        pltpu.VMEM((ENT_SL,), f32),
        pltpu.VMEM((16,), f32),
        pltpu.VMEM((16,), i32),
        pltpu.VMEM((16,), f32),
        pltpu.VMEM((16,), f32),
        pltpu.SemaphoreType.DMA,
        pltpu.SemaphoreType.DMA,
        pltpu.VMEM_SHARED((ENT_PAD,), f32),
        pltpu.VMEM_SHARED((ENT_PAD,), f32),
        pltpu.VMEM_SHARED((ENT_PAD,), f32),
        pltpu.VMEM_SHARED((ENT_PAD,), f32),
        pltpu.VMEM_SHARED((ENT_PAD,), f32),
        pltpu.VMEM_SHARED((REL_PAD,), f32),
        pltpu.VMEM_SHARED((REL_PAD,), f32),
        pltpu.VMEM_SHARED((REL_PAD,), f32),
        pltpu.VMEM_SHARED((REL_PAD,), f32),
        pltpu.VMEM_SHARED((REL_PAD,), f32),
        pltpu.VMEM_SHARED((16,), f32),
        pltpu.VMEM_SHARED((16,), f32),
    ]
    out_type = (
        jax.ShapeDtypeStruct((BSZ * ENT_PAD,), f32),
        jax.ShapeDtypeStruct((BSZ * REL_PAD,), f32),
    )
    ent_out, rel_out = pl.kernel(
        _sc_body, out_type=out_type, mesh=mesh, scratch_types=scratch,
    )(eterm.reshape(-1), rterm.reshape(-1), sub_p, rel_p, obj_p, valid_p,
      keyi.reshape(-1), keyv.reshape(-1), wv.reshape(-1))
    return ent_out.reshape(BSZ, ENT_PAD), rel_out.reshape(BSZ, REL_PAD)


@jax.jit
def kernel(text_emb, q_word, edge_sub, edge_rel, edge_obj, keyconcepts,
           ent_table, rel_table, step_W, step_b,
           rel_cls_w, rel_cls_b, ent_cls_w, ent_cls_b,
           type_cls_w, type_cls_b, hop_w, hop_b, anstype_w, anstype_b):
    f32 = jnp.float32

    cq = jnp.tanh(jnp.einsum("bd,tde->bte", text_emb, step_W) + step_b[None])
    q_logits = jnp.einsum("bte,ble->btl", cq, q_word)
    q_dist = jax.nn.softmax(q_logits, axis=2)
    q_dist = q_dist / (jnp.sum(q_dist, axis=2, keepdims=True) + 1e-6)
    ctx = jnp.einsum("btl,ble->bte", q_dist, q_word) + cq

    type_score = jax.nn.sigmoid(jnp.einsum("bte,e->bt", ctx, type_cls_w)
                                + type_cls_b)
    hop_attn = jax.nn.softmax(text_emb @ hop_w + hop_b, axis=1)
    anstype_attn = jax.nn.softmax(text_emb @ anstype_w + anstype_b, axis=1)

    vt_e = (ctx * ent_cls_w[None, None, :]).reshape(BSZ * NUM_STEPS, DIM)
    vt_r = (ctx * rel_cls_w[None, None, :]).reshape(BSZ * NUM_STEPS, DIM)
    pad2 = jnp.zeros((8 - BSZ * NUM_STEPS, DIM), f32)
    vt_e = jnp.concatenate([vt_e, pad2], axis=0)
    vt_r = jnp.concatenate([vt_r, pad2], axis=0)
    ts = type_score.reshape(BSZ * NUM_STEPS)
    pad1 = jnp.zeros((8 - BSZ * NUM_STEPS,), f32)
    scale_e = jnp.concatenate([1.0 - ts, pad1]).reshape(8, 1)
    scale_r = jnp.concatenate([ts, pad1]).reshape(8, 1)
    bias_e = jnp.full((8, 1), ent_cls_b, f32)
    bias_r = jnp.full((8, 1), rel_cls_b, f32)

    went = hop_attn * anstype_attn[:, 0:1]
    wrel = hop_attn * anstype_attn[:, 1:2]
    wv = jnp.concatenate([went, wrel, jnp.zeros((BSZ, 16 - 2 * NUM_STEPS))],
                         axis=1).astype(f32)

    eterm = _tc_prob_tables(ent_table, vt_e, scale_e, bias_e, ENT_PAD)
    rterm = _tc_prob_tables(rel_table, vt_r, scale_r, bias_r, REL_PAD)

    i32 = jnp.int32
    def pad_edges(x):
        return jnp.pad(x.astype(i32), ((0, 0), (0, E_PAD - E))
                       ).reshape(BSZ, E_PAD // 128, 128)
    sub_p = pad_edges(edge_sub)
    rel_p = pad_edges(edge_rel)
    obj_p = pad_edges(edge_obj)
    valid = (jnp.arange(E_PAD) < E).astype(f32)
    valid_p = jnp.broadcast_to(valid, (BSZ, E_PAD)).reshape(
        BSZ, E_PAD // 128, 128)

    keyi = jnp.pad(keyconcepts.astype(i32), ((0, 0), (0, 16 - K)))
    keyv = jnp.broadcast_to(
        jnp.where(jnp.arange(16) < K, jnp.float32(1.0 / K), 0.0), (BSZ, 16))

    ent_out, rel_out = _sc_traverse(eterm, rterm, sub_p, rel_p, obj_p,
                                    valid_p, keyi, keyv, wv)

    return jnp.concatenate([ent_out[:, :NUM_ENT], rel_out[:, :NUM_REL]],
                           axis=1)


# speedup vs baseline: 1.0451x; 1.0192x over previous
import functools
import jax
import jax.numpy as jnp
from jax import lax
from jax.experimental import pallas as pl
from jax.experimental.pallas import tpu as pltpu
from jax.experimental.pallas import tpu_sc as plsc

NUM_ENT = 50000
NUM_REL = 500
NUM_STEPS = 3
DIM = 768
BSZ = 2
E = 16000
K = 4

N_TILES = 16
ENT_PAD = 50176
ENT_SL = ENT_PAD // N_TILES
REL_PAD = 512
E_PAD = 16384
EPT = E_PAD // N_TILES
NJ = EPT // 128
ROW_BLK = 3584


def _tc_prob_body(vt_ref, scale_ref, bias_ref, x_ref, o_ref):
    x = x_ref[...]
    vt = vt_ref[...]
    logits = jnp.einsum("kd,nd->kn", vt, x,
                        preferred_element_type=jnp.float32)
    logits = logits + bias_ref[...]
    o_ref[...] = scale_ref[...] * jax.nn.sigmoid(logits)


def _tc_prob_tables(table, vt, scale, bias, n_out):
    n_rows = table.shape[0]
    blk = min(ROW_BLK, n_out)
    grid = n_out // blk
    assert grid * blk == n_out and n_out >= n_rows
    return pl.pallas_call(
        _tc_prob_body,
        grid=(grid,),
        in_specs=[
            pl.BlockSpec((8, DIM), lambda i: (0, 0)),
            pl.BlockSpec((8, 1), lambda i: (0, 0)),
            pl.BlockSpec((8, 1), lambda i: (0, 0)),
            pl.BlockSpec((blk, DIM), lambda i: (i, 0)),
        ],
        out_specs=pl.BlockSpec((8, blk), lambda i: (0, i)),
        out_shape=jax.ShapeDtypeStruct((8, n_out), jnp.float32),
    )(vt, scale, bias, table)


def _sc_body(eterm_h, rterm_h, sub_h, rel_h, obj_h, valid_h,
             keyi_h, keyv_h, wv_h,
             ent_out, rel_out,
             zero_v, sub_iv, rel_iv, obj_iv, valid_v,
             ss_v, ep_v, rp_v, contrib_v, tprob_v,
             rwork_v, frel_v,
             eslice_v, norm_v, fent_v,
             wv_v, keyi_v, keyv_v, padd_v,
             gsem, ssem, psem,
             laste_sp, enew_sp, eterm0_sp, eterm1_sp, eterm2_sp,
             rterm0_sp, rterm1_sp, rterm2_sp,
             rnew0_sp, rnew1_sp, partials0_sp, partials1_sp):
    eterm_sps = (eterm0_sp, eterm1_sp, eterm2_sp)
    rterm_sps = (rterm0_sp, rterm1_sp, rterm2_sp)
    rnew_sps = (rnew0_sp, rnew1_sp)
    partials_sps = (partials0_sp, partials1_sp)
    cid = lax.axis_index("c")
    tid = lax.axis_index("s")
    b = cid
    ent_lo = tid * ENT_SL
    lane = lax.iota(jnp.int32, 16)

    def al8(x):
        return pl.multiple_of(x, 8)

    def hsum(v):
        s = v[0]
        for i in range(1, 16):
            s = s + v[i]
        return s

    edge_descs = [
        pltpu.async_copy(sub_h.at[b, pl.ds(tid * NJ, NJ), :], sub_iv, gsem),
        pltpu.async_copy(rel_h.at[b, pl.ds(tid * NJ, NJ), :], rel_iv, gsem),
        pltpu.async_copy(obj_h.at[b, pl.ds(tid * NJ, NJ), :], obj_iv, gsem),
        pltpu.async_copy(valid_h.at[b, pl.ds(tid * NJ, NJ), :], valid_v,
                         gsem),
        pltpu.async_copy(wv_h.at[pl.ds(al8(b * 16), 16)], wv_v, gsem),
    ]
    bounce = (eslice_v, norm_v, fent_v)
    et_descs = [
        pltpu.async_copy(
            eterm_h.at[pl.ds(al8((b * NUM_STEPS + t) * ENT_PAD + ent_lo),
                             ENT_SL)],
            bounce[t], ssem)
        for t in range(NUM_STEPS)
    ]

    def _zz(i, _):
        zero_v[pl.ds(i * 16, 16)] = jnp.zeros((16,), jnp.float32)
        return 0
    lax.fori_loop(0, ENT_SL // 16, _zz, 0)

    for t in range(NUM_STEPS):
        et_descs[t].wait()
        pltpu.sync_copy(bounce[t], eterm_sps[t].at[pl.ds(ent_lo, ENT_SL)])

    def _z196(i, _):
        fent_v[pl.ds(i * 16, 16)] = jnp.zeros((16,), jnp.float32)
        return 0
    lax.fori_loop(0, ENT_SL // 16, _z196, 0)

    def _z32(i, _):
        frel_v[pl.ds(i * 16, 16)] = jnp.zeros((16,), jnp.float32)
        return 0
    lax.fori_loop(0, REL_PAD // 16, _z32, 0)

    pltpu.sync_copy(zero_v, laste_sp.at[pl.ds(ent_lo, ENT_SL)])
    pltpu.sync_copy(zero_v, enew_sp.at[pl.ds(ent_lo, ENT_SL)])

    @pl.when(tid == 0)
    def _():
        for p in range(2):
            pltpu.sync_copy(zero_v.at[pl.ds(0, REL_PAD)], rnew_sps[p])
            pltpu.sync_copy(zero_v.at[pl.ds(0, 16)], partials_sps[p])
        for t in range(NUM_STEPS):
            pltpu.sync_copy(
                rterm_h.at[pl.ds(al8((b * NUM_STEPS + t) * REL_PAD),
                                 REL_PAD)],
                rwork_v)
            pltpu.sync_copy(rwork_v, rterm_sps[t])

    for d in edge_descs:
        d.wait()
    plsc.subcore_barrier()

    def fire_pf(t):
        ds_ = []
        for j in range(NJ):
            ds_.append(pltpu.async_copy(
                eterm_sps[t].at[obj_iv.at[j]], ep_v.at[j], psem))
            ds_.append(pltpu.async_copy(
                rterm_sps[t].at[rel_iv.at[j]], rp_v.at[j], psem))
        return ds_

    pf_descs = fire_pf(0)

    @pl.when(tid == 0)
    def _():
        pltpu.sync_copy(keyi_h.at[pl.ds(al8(b * 16), 16)], keyi_v)
        pltpu.sync_copy(keyv_h.at[pl.ds(al8(b * 16), 16)], keyv_v)
        pltpu.sync_copy(keyv_v, laste_sp.at[keyi_v], add=True)

    margin = jnp.float32(1.0 / K - 1e-6)

    for t in range(NUM_STEPS):
        p = t % 2
        plsc.subcore_barrier()

        g_descs = []
        for j in range(NJ):
            g_descs.append(pltpu.async_copy(
                laste_sp.at[sub_iv.at[j]], ss_v.at[j], gsem))
        for d in pf_descs:
            d.wait()
        for d in g_descs:
            d.wait()

        def _chunk(c, _):
            j = c // 8
            sl = pl.ds((c % 8) * 16, 16)
            rp = rp_v[j, sl]
            ss16 = ss_v[j, sl]
            ep16 = ep_v[j, sl]
            va = valid_v[j, sl]
            tp = (rp + ep16) * va
            tp = jnp.where(ss16 > margin, tp, jnp.float32(0.0))
            tprob_v[j, sl] = tp
            contrib_v[j, sl] = ss16 * tp
            return 0
        lax.fori_loop(0, EPT // 16, _chunk, 0)

        if t + 1 < NUM_STEPS:
            pf_descs = fire_pf(t + 1)

        s_descs = []
        for j in range(NJ):
            s_descs.append(pltpu.async_copy(
                contrib_v.at[j], enew_sp.at[obj_iv.at[j]], ssem, add=True))
            s_descs.append(pltpu.async_copy(
                tprob_v.at[j], rnew_sps[p].at[rel_iv.at[j]], ssem, add=True))
        for d in s_descs:
            d.wait()

        plsc.subcore_barrier()

        rw_d = pltpu.async_copy(rnew_sps[p], rwork_v, gsem)
        pltpu.sync_copy(enew_sp.at[pl.ds(ent_lo, ENT_SL)], eslice_v)

        def _acc(i, a):
            return a + jnp.abs(eslice_v[pl.ds(i * 16, 16)])
        acc = lax.fori_loop(0, ENT_SL // 16, _acc,
                            jnp.zeros((16,), jnp.float32))
        partial = hsum(acc)
        padd_v[...] = jnp.where(lane == tid, partial, jnp.float32(0.0))
        pltpu.sync_copy(padd_v, partials_sps[p].at[lane], add=True)

        pltpu.sync_copy(zero_v, enew_sp.at[pl.ds(ent_lo, ENT_SL)])
        rw_d.wait()
        def _racc(i, a):
            return a + jnp.abs(rwork_v[pl.ds(i * 16, 16)])
        racc = lax.fori_loop(0, REL_PAD // 16, _racc,
                             jnp.zeros((16,), jnp.float32))
        tot_r = jnp.maximum(hsum(racc), jnp.float32(1e-12))

        plsc.subcore_barrier()

        tot_e = hsum(partials_sp_read(wv_v, partials_sps[p], padd_v))
        tot_e = jnp.maximum(tot_e, jnp.float32(1e-12))

        wvec = wv_v[...]
        went = wvec[t]
        wrel = wvec[NUM_STEPS + t]

        def _norm_e(i, _):
            sl = pl.ds(i * 16, 16)
            c = eslice_v[sl] / tot_e
            norm_v[sl] = c
            fent_v[sl] = fent_v[sl] + went * c
            return 0
        lax.fori_loop(0, ENT_SL // 16, _norm_e, 0)

        def _norm_r(i, _):
            sl = pl.ds(i * 16, 16)
            frel_v[sl] = frel_v[sl] + wrel * (rwork_v[sl] / tot_r)
            return 0
        lax.fori_loop(0, REL_PAD // 16, _norm_r, 0)

        pltpu.sync_copy(norm_v, laste_sp.at[pl.ds(ent_lo, ENT_SL)])

        @pl.when(tid == 0)
        def _():
            pltpu.sync_copy(zero_v.at[pl.ds(0, REL_PAD)], rnew_sps[1 - p])
            pltpu.sync_copy(zero_v.at[pl.ds(0, 16)], partials_sps[1 - p])

    pltpu.sync_copy(fent_v, ent_out.at[pl.ds(al8(b * ENT_PAD + ent_lo),
                                             ENT_SL)])

    @pl.when(tid == 0)
    def _():
        pltpu.sync_copy(frel_v, rel_out.at[pl.ds(al8(b * REL_PAD), REL_PAD)])


def partials_sp_read(wv_v, partials_sp, padd_v):
    pltpu.sync_copy(partials_sp, padd_v)
    return padd_v[...]


def _sc_traverse(eterm, rterm, sub_p, rel_p, obj_p, valid_p, keyi, keyv, wv):
    mesh = plsc.VectorSubcoreMesh(core_axis_name="c", subcore_axis_name="s")
    f32, i32 = jnp.float32, jnp.int32
    scratch = [
        pltpu.VMEM((ENT_SL,), f32),
        pltpu.VMEM((NJ, 128), i32),
        pltpu.VMEM((NJ, 128), i32),
        pltpu.VMEM((NJ, 128), i32),
        pltpu.VMEM((NJ, 128), f32),
        pltpu.VMEM((NJ, 128), f32),
        pltpu.VMEM((NJ, 128), f32),
        pltpu.VMEM((NJ, 128), f32),
        pltpu.VMEM((NJ, 128), f32),
        pltpu.VMEM((NJ, 128), f32),
        pltpu.VMEM((REL_PAD,), f32),
        pltpu.VMEM((REL_PAD,), f32),
        pltpu.VMEM((ENT_SL,), f32),
        pltpu.VMEM((ENT_SL,), f32),
        pltpu.VMEM((ENT_SL,), f32),
        pltpu.VMEM((16,), f32),
        pltpu.VMEM((16,), i32),
        pltpu.VMEM((16,), f32),
        pltpu.VMEM((16,), f32),
        pltpu.SemaphoreType.DMA,
        pltpu.SemaphoreType.DMA,
        pltpu.SemaphoreType.DMA,
        pltpu.VMEM_SHARED((ENT_PAD,), f32),
        pltpu.VMEM_SHARED((ENT_PAD,), f32),
        pltpu.VMEM_SHARED((ENT_PAD,), f32),
        pltpu.VMEM_SHARED((ENT_PAD,), f32),
        pltpu.VMEM_SHARED((ENT_PAD,), f32),
        pltpu.VMEM_SHARED((REL_PAD,), f32),
        pltpu.VMEM_SHARED((REL_PAD,), f32),
        pltpu.VMEM_SHARED((REL_PAD,), f32),
        pltpu.VMEM_SHARED((REL_PAD,), f32),
        pltpu.VMEM_SHARED((REL_PAD,), f32),
        pltpu.VMEM_SHARED((16,), f32),
        pltpu.VMEM_SHARED((16,), f32),
    ]
    out_type = (
        jax.ShapeDtypeStruct((BSZ * ENT_PAD,), f32),
        jax.ShapeDtypeStruct((BSZ * REL_PAD,), f32),
    )
    ent_out, rel_out = pl.kernel(
        _sc_body, out_type=out_type, mesh=mesh, scratch_types=scratch,
    )(eterm.reshape(-1), rterm.reshape(-1), sub_p, rel_p, obj_p, valid_p,
      keyi.reshape(-1), keyv.reshape(-1), wv.reshape(-1))
    return ent_out.reshape(BSZ, ENT_PAD), rel_out.reshape(BSZ, REL_PAD)


@jax.jit
def kernel(text_emb, q_word, edge_sub, edge_rel, edge_obj, keyconcepts,
           ent_table, rel_table, step_W, step_b,
           rel_cls_w, rel_cls_b, ent_cls_w, ent_cls_b,
           type_cls_w, type_cls_b, hop_w, hop_b, anstype_w, anstype_b):
    f32 = jnp.float32

    cq = jnp.tanh(jnp.einsum("bd,tde->bte", text_emb, step_W) + step_b[None])
    q_logits = jnp.einsum("bte,ble->btl", cq, q_word)
    q_dist = jax.nn.softmax(q_logits, axis=2)
    q_dist = q_dist / (jnp.sum(q_dist, axis=2, keepdims=True) + 1e-6)
    ctx = jnp.einsum("btl,ble->bte", q_dist, q_word) + cq

    type_score = jax.nn.sigmoid(jnp.einsum("bte,e->bt", ctx, type_cls_w)
                                + type_cls_b)
    hop_attn = jax.nn.softmax(text_emb @ hop_w + hop_b, axis=1)
    anstype_attn = jax.nn.softmax(text_emb @ anstype_w + anstype_b, axis=1)

    vt_e = (ctx * ent_cls_w[None, None, :]).reshape(BSZ * NUM_STEPS, DIM)
    vt_r = (ctx * rel_cls_w[None, None, :]).reshape(BSZ * NUM_STEPS, DIM)
    pad2 = jnp.zeros((8 - BSZ * NUM_STEPS, DIM), f32)
    vt_e = jnp.concatenate([vt_e, pad2], axis=0)
    vt_r = jnp.concatenate([vt_r, pad2], axis=0)
    ts = type_score.reshape(BSZ * NUM_STEPS)
    pad1 = jnp.zeros((8 - BSZ * NUM_STEPS,), f32)
    scale_e = jnp.concatenate([1.0 - ts, pad1]).reshape(8, 1)
    scale_r = jnp.concatenate([ts, pad1]).reshape(8, 1)
    bias_e = jnp.full((8, 1), ent_cls_b, f32)
    bias_r = jnp.full((8, 1), rel_cls_b, f32)

    went = hop_attn * anstype_attn[:, 0:1]
    wrel = hop_attn * anstype_attn[:, 1:2]
    wv = jnp.concatenate([went, wrel, jnp.zeros((BSZ, 16 - 2 * NUM_STEPS))],
                         axis=1).astype(f32)

    eterm = _tc_prob_tables(ent_table, vt_e, scale_e, bias_e, ENT_PAD)
    rterm = _tc_prob_tables(rel_table, vt_r, scale_r, bias_r, REL_PAD)

    i32 = jnp.int32
    def pad_edges(x):
        return jnp.pad(x.astype(i32), ((0, 0), (0, E_PAD - E))
                       ).reshape(BSZ, E_PAD // 128, 128)
    sub_p = pad_edges(edge_sub)
    rel_p = pad_edges(edge_rel)
    obj_p = pad_edges(edge_obj)
    valid = (jnp.arange(E_PAD) < E).astype(f32)
    valid_p = jnp.broadcast_to(valid, (BSZ, E_PAD)).reshape(
        BSZ, E_PAD // 128, 128)

    keyi = jnp.pad(keyconcepts.astype(i32), ((0, 0), (0, 16 - K)))
    keyv = jnp.broadcast_to(
        jnp.where(jnp.arange(16) < K, jnp.float32(1.0 / K), 0.0), (BSZ, 16))

    ent_out, rel_out = _sc_traverse(eterm, rterm, sub_p, rel_p, obj_p,
                                    valid_p, keyi, keyv, wv)

    return jnp.concatenate([ent_out[:, :NUM_ENT], rel_out[:, :NUM_REL]],
                           axis=1)
